# baseline (speedup 1.0000x reference)
import functools

import jax
import jax.numpy as jnp
from jax import lax
from jax.experimental import pallas as pl
from jax.experimental.pallas import tpu as pltpu
from jax.experimental.pallas import tpu_sc as plsc

VOCAB = 100000
EMBED = 64
BATCH = 4096
HIST = 200

NC = 2
NS = 16
LANES = 16
NW = NC * NS
BPW = BATCH // NW
HALF = HIST // 2
NVEC = EMBED // LANES

_mesh = plsc.VectorSubcoreMesh(core_axis_name="c", subcore_axis_name="s")


@functools.partial(
    pl.kernel,
    out_type=jax.ShapeDtypeStruct((BATCH, EMBED), jnp.float32),
    mesh=_mesh,
    compiler_params=pltpu.CompilerParams(use_tc_tiling_on_sc=False),
    scratch_types=[
        pltpu.VMEM((2 * BPW, HALF), jnp.int32),
        pltpu.VMEM((6, HIST, EMBED), jnp.float32),
        pltpu.VMEM((BPW, EMBED), jnp.float32),
        pltpu.SemaphoreType.DMA,
        pltpu.SemaphoreType.DMA,
        pltpu.SemaphoreType.DMA,
        pltpu.SemaphoreType.DMA,
        pltpu.SemaphoreType.DMA,
        pltpu.SemaphoreType.DMA,
    ],
)
def _embed_pool(x_hbm, table_hbm, dummy_hbm, out_hbm, idx_v, rows_v, out_v,
                sem0, sem1, sem2, sem3, sem4, sem5):
    sems = (sem0, sem1, sem2, sem3, sem4, sem5)
    NBUF = 6
    wid = lax.axis_index("s") * NC + lax.axis_index("c")
    base = wid * BPW

    pltpu.sync_copy(x_hbm.at[pl.ds(2 * base, 2 * BPW)], idx_v)

    def issue(b, buf):
        pltpu.async_copy(table_hbm.at[idx_v.at[2 * b]],
                         rows_v.at[buf, pl.ds(0, HALF)], sems[buf])
        pltpu.async_copy(table_hbm.at[idx_v.at[2 * b + 1]],
                         rows_v.at[buf, pl.ds(HALF, HALF)], sems[buf])

    def wait(buf):
        pltpu.make_async_copy(dummy_hbm, rows_v.at[buf], sems[buf]).wait()

    def reduce(b, buf):
        U = 8

        def body(t, acc):
            acc = list(acc)
            for u in range(U):
                c = NVEC * (u % 2)
                for i in range(NVEC):
                    acc[c + i] = acc[c + i] + rows_v[
                        buf, t * U + u, pl.ds(LANES * i, LANES)]
            return tuple(acc)

        acc = lax.fori_loop(
            0, HIST // U, body,
            tuple(jnp.zeros((LANES,), jnp.float32) for _ in range(2 * NVEC)))
        for i in range(NVEC):
            out_v[b, pl.ds(LANES * i, LANES)] = (
                (acc[i] + acc[NVEC + i]) * (1.0 / HIST))

    for b in range(NBUF - 1):
        issue(b, b)

    n_groups = -(-BPW // NBUF)

    def outer(j, _):
        b0 = j * NBUF
        for u in range(NBUF):
            nb = b0 + u + NBUF - 1

            @pl.when(nb < BPW)
            def _():
                issue(nb, (u + NBUF - 1) % NBUF)

            @pl.when(b0 + u < BPW)
            def _():
                wait(u)
                reduce(b0 + u, u)
        return 0

    lax.fori_loop(0, n_groups, outer, 0)
    pltpu.sync_copy(out_v, out_hbm.at[pl.ds(base, BPW)])


def kernel(x, table):
    x2 = x.astype(jnp.int32).reshape(2 * BATCH, HALF)
    dummy = jnp.zeros((HIST, EMBED), jnp.float32)
    return _embed_pool(x2, table, dummy)


# speedup vs baseline: 1.0075x; 1.0075x over previous
import functools

import jax
import jax.numpy as jnp
from jax import lax
from jax.experimental import pallas as pl
from jax.experimental.pallas import tpu as pltpu
from jax.experimental.pallas import tpu_sc as plsc

VOCAB = 100000
EMBED = 64
BATCH = 4096
HIST = 200

NC = 2
NS = 16
LANES = 16
NW = NC * NS
BPW = BATCH // NW
HALF = HIST // 2
NVEC = EMBED // LANES

CB = 2
NBUF = 3
NENT = BPW // CB

_mesh = plsc.VectorSubcoreMesh(core_axis_name="c", subcore_axis_name="s")


@functools.partial(
    pl.kernel,
    out_type=jax.ShapeDtypeStruct((BATCH, EMBED), jnp.float32),
    mesh=_mesh,
    compiler_params=pltpu.CompilerParams(use_tc_tiling_on_sc=False),
    scratch_types=[
        pltpu.VMEM((2 * BPW, HALF), jnp.int32),
        pltpu.VMEM((NBUF, CB * HIST, EMBED), jnp.float32),
        pltpu.VMEM((BPW, EMBED), jnp.float32),
        pltpu.SemaphoreType.DMA,
        pltpu.SemaphoreType.DMA,
        pltpu.SemaphoreType.DMA,
    ],
)
def _embed_pool(x_hbm, table_hbm, dummy_hbm, out_hbm, idx_v, rows_v, out_v,
                sem0, sem1, sem2):
    sems = (sem0, sem1, sem2)
    wid = lax.axis_index("s") * NC + lax.axis_index("c")
    base = wid * BPW

    pltpu.sync_copy(x_hbm.at[pl.ds(2 * base, 2 * BPW)], idx_v)

    def issue(e, buf):
        for g in range(2 * CB):
            pltpu.async_copy(table_hbm.at[idx_v.at[2 * CB * e + g]],
                             rows_v.at[buf, pl.ds(g * HALF, HALF)], sems[buf])

    def wait(buf):
        pltpu.make_async_copy(dummy_hbm, rows_v.at[buf], sems[buf]).wait()

    def reduce(b, buf, c):
        U = 8

        def body(t, acc):
            acc = list(acc)
            for u in range(U):
                ch = NVEC * (u % 2)
                for i in range(NVEC):
                    acc[ch + i] = acc[ch + i] + rows_v[
                        buf, c * HIST + t * U + u, pl.ds(LANES * i, LANES)]
            return tuple(acc)

        acc = lax.fori_loop(
            0, HIST // U, body,
            tuple(jnp.zeros((LANES,), jnp.float32) for _ in range(2 * NVEC)))
        for i in range(NVEC):
            out_v[b, pl.ds(LANES * i, LANES)] = (
                (acc[i] + acc[NVEC + i]) * (1.0 / HIST))

    for e in range(NBUF - 1):
        issue(e, e)

    n_groups = -(-NENT // NBUF)

    def outer(j, _):
        e0 = j * NBUF
        for u in range(NBUF):
            ne = e0 + u + NBUF - 1

            @pl.when(ne < NENT)
            def _():
                issue(ne, (u + NBUF - 1) % NBUF)

            @pl.when(e0 + u < NENT)
            def _():
                wait(u)
                for c in range(CB):
                    reduce((e0 + u) * CB + c, u, c)
        return 0

    lax.fori_loop(0, n_groups, outer, 0)
    pltpu.sync_copy(out_v, out_hbm.at[pl.ds(base, BPW)])


def kernel(x, table):
    x2 = x.astype(jnp.int32).reshape(2 * BATCH, HALF)
    dummy = jnp.zeros((CB * HIST, EMBED), jnp.float32)
    return _embed_pool(x2, table, dummy)


# speedup vs baseline: 1.0200x; 1.0124x over previous
import functools

import jax
import jax.numpy as jnp
from jax import lax
from jax.experimental import pallas as pl
from jax.experimental.pallas import tpu as pltpu
from jax.experimental.pallas import tpu_sc as plsc

VOCAB = 100000
EMBED = 64
BATCH = 4096
HIST = 200

NC = 2
NS = 16
LANES = 16
NW = NC * NS
BPW = BATCH // NW
HALF = HIST // 2
NVEC = EMBED // LANES

CB = 2
NBUF = 3
NENT = BPW // CB

_mesh = plsc.VectorSubcoreMesh(core_axis_name="c", subcore_axis_name="s")


@functools.partial(
    pl.kernel,
    out_type=jax.ShapeDtypeStruct((BATCH, EMBED), jnp.float32),
    mesh=_mesh,
    compiler_params=pltpu.CompilerParams(use_tc_tiling_on_sc=False),
    scratch_types=[
        pltpu.VMEM((NENT * CB * HIST,), jnp.int32),
        pltpu.VMEM((NBUF, CB * HIST, EMBED), jnp.float32),
        pltpu.VMEM((BPW, EMBED), jnp.float32),
        pltpu.SemaphoreType.DMA,
        pltpu.SemaphoreType.DMA,
        pltpu.SemaphoreType.DMA,
    ],
)
def _embed_pool(x_hbm, table_hbm, dummy_hbm, out_hbm, idx_v, rows_v, out_v,
                sem0, sem1, sem2):
    sems = (sem0, sem1, sem2)
    wid = lax.axis_index("s") * NC + lax.axis_index("c")
    base = wid * BPW

    pltpu.sync_copy(x_hbm.at[pl.ds(wid * BPW * HIST, BPW * HIST)], idx_v)

    def issue(e, buf):
        pltpu.async_copy(table_hbm.at[idx_v.at[pl.ds(e * CB * HIST, CB * HIST)]],
                         rows_v.at[buf], sems[buf])

    def wait(buf):
        pltpu.make_async_copy(dummy_hbm, rows_v.at[buf], sems[buf]).wait()

    def reduce(b, buf, c):
        U = 8

        def body(t, acc):
            acc = list(acc)
            for u in range(U):
                ch = NVEC * (u % 2)
                for i in range(NVEC):
                    acc[ch + i] = acc[ch + i] + rows_v[
                        buf, c * HIST + t * U + u, pl.ds(LANES * i, LANES)]
            return tuple(acc)

        acc = lax.fori_loop(
            0, HIST // U, body,
            tuple(jnp.zeros((LANES,), jnp.float32) for _ in range(2 * NVEC)))
        for i in range(NVEC):
            out_v[b, pl.ds(LANES * i, LANES)] = (
                (acc[i] + acc[NVEC + i]) * (1.0 / HIST))

    for e in range(NBUF - 1):
        issue(e, e)

    n_groups = -(-NENT // NBUF)

    def outer(j, _):
        e0 = j * NBUF
        for u in range(NBUF):
            ne = e0 + u + NBUF - 1

            @pl.when(ne < NENT)
            def _():
                issue(ne, (u + NBUF - 1) % NBUF)

            @pl.when(e0 + u < NENT)
            def _():
                wait(u)
                for c in range(CB):
                    reduce((e0 + u) * CB + c, u, c)
        return 0

    lax.fori_loop(0, n_groups, outer, 0)
    pltpu.sync_copy(out_v, out_hbm.at[pl.ds(base, BPW)])


def kernel(x, table):
    x3 = x.astype(jnp.int32).reshape(BATCH * HIST)
    dummy = jnp.zeros((CB * HIST, EMBED), jnp.float32)
    return _embed_pool(x3, table, dummy)
